# baseline (speedup 1.0000x reference)
import functools

import jax
import jax.numpy as jnp
from jax import lax
from jax.experimental import pallas as pl
from jax.experimental.pallas import tpu as pltpu
from jax.experimental.pallas import tpu_sc as plsc

VOCAB = 1000000
D = 32
B = 4096
H = 200

NC = 2
NS = 16
NW = NC * NS

NCOL = VOCAB // 128
COLS_PER_W = NCOL // NW
COL_EXTRA = NCOL - COLS_PER_W * NW
TAIL = VOCAB - NCOL * 128
NLINE = VOCAB // 4

NBLK = B // 128
NBUF = 4
WIN = 8

_TILED = pltpu.CompilerParams(
    use_tc_tiling_on_sc=True, needs_layout_passes=False
)
_LINEAR = pltpu.CompilerParams(
    use_tc_tiling_on_sc=False, needs_layout_passes=False
)
_MESH = plsc.VectorSubcoreMesh(core_axis_name="c", subcore_axis_name="s")


def _windowed(ops):
    pend = []
    for emit, commit in ops:
        pend.append((commit, emit()))
        if len(pend) > WIN:
            c, v = pend.pop(0)
            c(v)
    for c, v in pend:
        c(v)


def _transpose_block(src_v, dst_v, iota, nlane):
    def ops():
        for r in range(nlane // 4):
            for q in range(4):
                col = jnp.full((16,), 4 * r + q, jnp.int32)
                for half in range(2):
                    def emit(col=col, half=half):
                        return plsc.load_gather(src_v, [iota + 16 * half, col])

                    def commit(v, r=r, q=q, half=half):
                        plsc.store_scatter(
                            dst_v.at[r], [iota + (q * 32 + 16 * half)], v
                        )

                    yield emit, commit

    _windowed(ops())


def _tab_body(tab_hbm, scr_hbm, in_v, out_v, tail_v, sem_ld, sem_st):
    wid = lax.axis_index("s") * NC + lax.axis_index("c")
    iota = lax.broadcasted_iota(jnp.int32, (16,), 0)

    def col_of(k):
        return wid + k * NW

    def issue_loads(c, par):
        for jb in range(4):
            pltpu.async_copy(
                tab_hbm.at[pl.ds(jb * 8, 8), pl.ds(c * 128, 128)],
                in_v.at[par, pl.ds(jb * 8, 8), :],
                sem_ld,
            )

    def wait_loads(c, par):
        for jb in range(4):
            pltpu.make_async_copy(
                tab_hbm.at[pl.ds(jb * 8, 8), pl.ds(c * 128, 128)],
                in_v.at[par, pl.ds(jb * 8, 8), :],
                sem_ld,
            ).wait()

    def store(c, par):
        pltpu.async_copy(
            out_v.at[par], scr_hbm.at[pl.ds(c * 32, 32), :], sem_st
        )

    def wait_store(c, par):
        pltpu.make_async_copy(
            out_v.at[par], scr_hbm.at[pl.ds(c * 32, 32), :], sem_st
        ).wait()

    issue_loads(col_of(0), 0)
    issue_loads(col_of(1), 1)

    def body(k2, carry):
        for par in range(2):
            k = 2 * k2 + par
            c = col_of(k)
            wait_loads(c, par)

            @pl.when(k2 > 0)
            def _():
                wait_store(c, par)

            _transpose_block(in_v.at[par], out_v.at[par], iota, 128)
            store(c, par)

            @pl.when(k2 < (COLS_PER_W // 2 - 1))
            def _():
                issue_loads(col_of(k + 2), par)

        return carry

    lax.fori_loop(0, COLS_PER_W // 2, body, 0)
    wait_store(col_of(COLS_PER_W - 2), 0)
    wait_store(col_of(COLS_PER_W - 1), 1)

    @pl.when(wid < COL_EXTRA)
    def _extra():
        c = COLS_PER_W * NW + wid
        issue_loads(c, 0)
        wait_loads(c, 0)
        _transpose_block(in_v.at[0], out_v.at[0], iota, 128)
        pltpu.sync_copy(out_v.at[0], scr_hbm.at[pl.ds(c * 32, 32), :])

    @pl.when(wid == NW - 1)
    def _tail():
        for jb in range(4):
            pltpu.async_copy(
                tab_hbm.at[pl.ds(jb * 8, 8), pl.ds(NCOL * 128, TAIL)],
                tail_v.at[pl.ds(jb * 8, 8), :],
                sem_ld,
            )
        for jb in range(4):
            pltpu.make_async_copy(
                tab_hbm.at[pl.ds(jb * 8, 8), pl.ds(NCOL * 128, TAIL)],
                tail_v.at[pl.ds(jb * 8, 8), :],
                sem_ld,
            ).wait()
        _transpose_block(tail_v, out_v.at[0], iota, TAIL)
        pltpu.sync_copy(
            out_v.at[0, pl.ds(0, TAIL // 4), :],
            scr_hbm.at[pl.ds(NCOL * 32, TAIL // 4), :],
        )


@functools.partial(
    pl.kernel,
    mesh=_MESH,
    out_type=jax.ShapeDtypeStruct((NLINE, 128), jnp.float32),
    scratch_types=[
        pltpu.VMEM((2, 32, 128), jnp.float32),
        pltpu.VMEM((2, 32, 128), jnp.float32),
        pltpu.VMEM((32, TAIL), jnp.float32),
        pltpu.SemaphoreType.DMA,
        pltpu.SemaphoreType.DMA,
    ],
    compiler_params=_TILED,
)
def _tab_relayout(tab_hbm, scr_hbm, in_v, out_v, tail_v, sem_ld, sem_st):
    _tab_body(tab_hbm, scr_hbm, in_v, out_v, tail_v, sem_ld, sem_st)


def _gather_body(
    xt_hbm, scr_hbm, out_hbm, idx_v, rows_v, tile_v, sem_g, sem_st
):
    wid = lax.axis_index("s") * NC + lax.axis_index("c")
    iota = lax.broadcasted_iota(jnp.int32, (16,), 0)

    pltpu.sync_copy(xt_hbm.at[:, wid, :, :], idx_v)

    def prefetch(t, buf):
        pltpu.async_copy(
            scr_hbm.at[idx_v.at[t // 8, t % 8]], rows_v.at[buf], sem_g
        )

    def wait_gather(buf):
        pltpu.make_async_copy(
            scr_hbm.at[pl.ds(0, 128), :], rows_v.at[buf], sem_g
        ).wait()

    def store_tile(t, par):
        pltpu.async_copy(
            tile_v.at[par],
            out_hbm.at[t, pl.ds(0, 4), wid, pl.ds(0, 8), :],
            sem_st,
        )

    def wait_store(t, par):
        pltpu.make_async_copy(
            tile_v.at[par],
            out_hbm.at[t, pl.ds(0, 4), wid, pl.ds(0, 8), :],
            sem_st,
        ).wait()

    def extract(buf, par):
        def ops():
            for jb in range(4):
                for s in range(8):
                    j = jb * 8 + s
                    col = jnp.full((16,), j, jnp.int32)
                    for l0 in range(8):
                        def emit(l0=l0, col=col):
                            return plsc.load_gather(
                                rows_v.at[buf], [iota + 16 * l0, col]
                            )

                        def commit(v, jb=jb, s=s, l0=l0):
                            plsc.store_scatter(
                                tile_v.at[par, jb, s], [iota + 16 * l0], v
                            )

                        yield emit, commit

        _windowed(ops())

    for b in range(NBUF):
        prefetch(b, b)

    def body(t2, carry):
        for par in range(NBUF):
            t = NBUF * t2 + par
            wait_gather(par)

            @pl.when(t2 > 0)
            def _():
                wait_store(t, par)

            extract(par, par)
            store_tile(t, par)

            @pl.when(t2 < (H // NBUF - 1))
            def _():
                prefetch(t + NBUF, par)

        return carry

    lax.fori_loop(0, H // NBUF, body, 0)
    for par in range(NBUF):
        wait_store(H - NBUF + par, par)


@functools.partial(
    pl.kernel,
    mesh=_MESH,
    out_type=jax.ShapeDtypeStruct((H, 4, NBLK, 8, 128), jnp.float32),
    scratch_types=[
        pltpu.VMEM((H // 8, 8, 128), jnp.int32),
        pltpu.VMEM((NBUF, 128, D), jnp.float32),
        pltpu.VMEM((NBUF, 4, 8, 128), jnp.float32),
        pltpu.SemaphoreType.DMA,
        pltpu.SemaphoreType.DMA,
    ],
    compiler_params=_LINEAR,
)
def _emb_gather(xt_hbm, scr_hbm, out_hbm, idx_v, rows_v, tile_v, sem_g, sem_st):
    _gather_body(xt_hbm, scr_hbm, out_hbm, idx_v, rows_v, tile_v, sem_g, sem_st)


def kernel(X, table):
    tab_t = table.T
    x4 = X.astype(jnp.int32).T.reshape(H // 8, 8, B // 128, 128)
    x4 = jnp.transpose(x4, (0, 2, 1, 3))
    scratch = _tab_relayout(tab_t)
    rows = scratch.reshape(VOCAB, D)
    out5 = _emb_gather(x4, rows)
    out = jnp.transpose(out5, (2, 4, 0, 1, 3))
    return out.reshape(B, H, D)


# speedup vs baseline: 1.3229x; 1.3229x over previous
import functools

import jax
import jax.numpy as jnp
from jax import lax
from jax.experimental import pallas as pl
from jax.experimental.pallas import tpu as pltpu
from jax.experimental.pallas import tpu_sc as plsc

VOCAB = 1000000
D = 32
B = 4096
H = 200
N = B * H

NC = 2
NS = 16
NW = NC * NS

NCHUNK = 4
CN = N // NCHUNK
PER_W = CN // NW
G = 128
NG = PER_W // G
K = 8
GROUP = K * G
NGROUP = PER_W // GROUP


def _emb_body(x_hbm, tab_hbm, out_hbm, idx_v, rows_v, sem):
    c = lax.axis_index("c")
    s = lax.axis_index("s")
    wid = s * NC + c
    pltpu.sync_copy(x_hbm.at[pl.ds(wid * NG, NG)], idx_v)
    out_base = wid * PER_W

    def group(g, carry):
        copies = []
        for k in range(K):
            cp = pltpu.async_copy(
                tab_hbm.at[idx_v.at[g * K + k]],
                rows_v.at[pl.ds(k * G, G)],
                sem,
            )
            copies.append(cp)
        for cp in copies:
            cp.wait()
        pltpu.sync_copy(rows_v, out_hbm.at[pl.ds(out_base + g * GROUP, GROUP)])
        return carry

    lax.fori_loop(0, NGROUP, group, 0)


@functools.partial(
    pl.kernel,
    mesh=plsc.VectorSubcoreMesh(core_axis_name="c", subcore_axis_name="s"),
    out_type=jax.ShapeDtypeStruct((CN, D), jnp.float32),
    scratch_types=[
        pltpu.VMEM((NG, G), jnp.int32),
        pltpu.VMEM((GROUP, D), jnp.float32),
        pltpu.SemaphoreType.DMA,
    ],
    compiler_params=pltpu.CompilerParams(use_tc_tiling_on_sc=False),
)
def _emb(x_hbm, tab_hbm, out_hbm, idx_v, rows_v, sem):
    _emb_body(x_hbm, tab_hbm, out_hbm, idx_v, rows_v, sem)


def kernel(X, table):
    xt = X.astype(jnp.int32).T
    hc = H // NCHUNK
    parts = []
    for i in range(NCHUNK):
        xi = lax.slice_in_dim(xt, i * hc, (i + 1) * hc)
        xi = xi.reshape(CN // G, G)
        oi = _emb(xi, table)
        parts.append(jnp.transpose(oi.reshape(hc, B, D), (1, 0, 2)))
    return jnp.concatenate(parts, axis=1)
